# baseline (speedup 1.0000x reference)
import functools

import jax
import jax.numpy as jnp
from jax import lax
from jax.experimental import pallas as pl
from jax.experimental.pallas import tpu as pltpu
from jax.experimental.pallas import tpu_sc as plsc

D_MODEL = 768
FF_DIM = 512
N_EXPERTS = 8
TOP_K = 2
N_TOK = 2048
BLK = 128
N_BLOCKS = 40
PADDED = N_BLOCKS * BLK


def _router_kernel(x_ref, wr_ref, pos0_ref, pos1_ref, w0_ref, w1_ref,
                   bexp_ref):
    x = x_ref[...]
    logits = lax.dot_general(x, wr_ref[...], (((1,), (1,)), ((), ())),
                             preferred_element_type=jnp.float32)
    col = lax.broadcasted_iota(jnp.int32, logits.shape, 1)
    m0 = jnp.max(logits, axis=1, keepdims=True)
    i0 = jnp.min(jnp.where(logits == m0, col, N_EXPERTS), axis=1,
                 keepdims=True)
    oh0 = col == i0
    masked = jnp.where(oh0, -jnp.inf, logits)
    m1 = jnp.max(masked, axis=1, keepdims=True)
    i1 = jnp.min(jnp.where(masked == m1, col, N_EXPERTS), axis=1,
                 keepdims=True)
    oh1 = col == i1
    w0 = 1.0 / (1.0 + jnp.exp(m1 - m0))
    w0_ref[...] = w0
    w1_ref[...] = 1.0 - w0

    oh0f = oh0.astype(jnp.float32)
    oh1f = oh1.astype(jnp.float32)
    oh01 = jnp.concatenate([oh0f, oh1f], axis=1).astype(jnp.bfloat16)
    ri = lax.broadcasted_iota(jnp.int32, (N_TOK, N_TOK), 0)
    ci = lax.broadcasted_iota(jnp.int32, (N_TOK, N_TOK), 1)
    tri = (ci < ri).astype(jnp.bfloat16)
    ex01 = lax.dot_general(tri, oh01, (((1,), (0,)), ((), ())),
                           preferred_element_type=jnp.float32)
    ex0 = ex01[:, :N_EXPERTS]
    ex1 = ex01[:, N_EXPERTS:]

    cnt0 = jnp.sum(oh0f, axis=0, keepdims=True)
    cnt = cnt0 + jnp.sum(oh1f, axis=0, keepdims=True)
    nb = lax.shift_right_logical(cnt.astype(jnp.int32) + (BLK - 1), 7)
    eri = lax.broadcasted_iota(jnp.int32, (N_EXPERTS, N_EXPERTS), 0)
    eci = lax.broadcasted_iota(jnp.int32, (N_EXPERTS, N_EXPERTS), 1)
    ustrict = (eri < eci).astype(jnp.float32)
    po = lax.dot_general(nb.astype(jnp.float32), ustrict,
                         (((1,), (0,)), ((), ())),
                         preferred_element_type=jnp.float32) * BLK

    pos0f = jnp.sum(jnp.where(oh0, po + ex0, 0.0), axis=1, keepdims=True)
    pos1f = jnp.sum(jnp.where(oh1, po + cnt0 + ex1, 0.0), axis=1,
                    keepdims=True)
    pos0_ref[...] = pos0f.astype(jnp.int32)
    pos1_ref[...] = pos1f.astype(jnp.int32)

    ones_col = jnp.ones((N_EXPERTS, 1), jnp.float32)
    outer = lax.dot_general(ones_col, po, (((1,), (0,)), ((), ())),
                            preferred_element_type=jnp.float32)
    eye = (eri == eci).astype(jnp.float32)
    po_col = jnp.sum(eye * outer, axis=1, keepdims=True)
    bcol = (lax.broadcasted_iota(jnp.int32, (N_EXPERTS, 64), 1)
            * BLK).astype(jnp.float32)
    cmp = (po_col <= bcol).astype(jnp.int32)
    bexp_ref[...] = jnp.clip(jnp.sum(cmp, axis=0, keepdims=True) - 1,
                             0, N_EXPERTS - 1)


def _router_call(flat, W_router):
    return pl.pallas_call(
        _router_kernel,
        out_shape=[
            jax.ShapeDtypeStruct((N_TOK, 1), jnp.int32),
            jax.ShapeDtypeStruct((N_TOK, 1), jnp.int32),
            jax.ShapeDtypeStruct((N_TOK, 1), jnp.float32),
            jax.ShapeDtypeStruct((N_TOK, 1), jnp.float32),
            jax.ShapeDtypeStruct((1, 64), jnp.int32),
        ],
    )(flat, W_router)


@functools.lru_cache(maxsize=None)
def _make_dispatch():
    mesh = plsc.VectorSubcoreMesh(core_axis_name="c", subcore_axis_name="s", num_cores=2, num_subcores=16)
    n_tiles = 32
    rows_per_tile = PADDED // n_tiles
    chunk = rows_per_tile // 2

    @functools.partial(
        pl.kernel,
        out_type=[
            jax.ShapeDtypeStruct((PADDED, D_MODEL), jnp.float32),
            jax.ShapeDtypeStruct((2 * PADDED,), jnp.int32),
            jax.ShapeDtypeStruct((2 * PADDED,), jnp.float32),
        ],
        mesh=mesh,
        scratch_types=[
            pltpu.VMEM((N_TOK,), jnp.int32),
            pltpu.VMEM((N_TOK,), jnp.int32),
            pltpu.VMEM((N_TOK,), jnp.float32),
            pltpu.VMEM((N_TOK,), jnp.float32),
            pltpu.VMEM((PADDED,), jnp.int32),
            pltpu.VMEM((PADDED,), jnp.float32),
            pltpu.VMEM((chunk,), jnp.int32),
            pltpu.VMEM((chunk, D_MODEL), jnp.float32),
            pltpu.SemaphoreType.DMA,
        ],
        compiler_params=pltpu.CompilerParams(needs_layout_passes=False),
    )
    def dispatch(x_hbm, pos0_hbm, pos1_hbm, w0_hbm, w1_hbm,
                 xs_hbm, stok_hbm, sw_hbm,
                 pos0_v, pos1_v, w0_v, w1_v, btok_v, bw_v, idx_v, rows_v,
                 sem):
        c = lax.axis_index("c")
        s = lax.axis_index("s")
        wid = s * 2 + c

        @pl.when(s == 0)
        def _scatter():
            pltpu.sync_copy(pos0_hbm, pos0_v)
            pltpu.sync_copy(pos1_hbm, pos1_v)
            pltpu.sync_copy(w0_hbm, w0_v)
            pltpu.sync_copy(w1_hbm, w1_v)

            zi = jnp.zeros((16,), jnp.int32)
            zf = jnp.zeros((16,), jnp.float32)
            for i in range(PADDED // 16):
                btok_v[pl.ds(i * 16, 16)] = zi
                bw_v[pl.ds(i * 16, 16)] = zf
            base_iota = lax.iota(jnp.int32, 16)
            for i in range(N_TOK // 16):
                idx = pos0_v[pl.ds(i * 16, 16)]
                plsc.store_scatter(btok_v, [idx], base_iota + i * 16)
                plsc.store_scatter(bw_v, [idx], w0_v[pl.ds(i * 16, 16)])
            for i in range(N_TOK // 16):
                idx = pos1_v[pl.ds(i * 16, 16)]
                plsc.store_scatter(btok_v, [idx], base_iota + i * 16)
                plsc.store_scatter(bw_v, [idx], w1_v[pl.ds(i * 16, 16)])

            pltpu.sync_copy(btok_v, stok_hbm.at[pl.ds(c * PADDED, PADDED)])
            pltpu.sync_copy(bw_v, sw_hbm.at[pl.ds(c * PADDED, PADDED)])

        plsc.subcore_barrier()

        base = wid * rows_per_tile
        for k in range(2):
            pltpu.sync_copy(stok_hbm.at[pl.ds(c * PADDED + base + k * chunk, chunk)],
                            idx_v)
            pltpu.async_copy(x_hbm.at[idx_v], rows_v, sem).wait()
            pltpu.sync_copy(rows_v, xs_hbm.at[pl.ds(base + k * chunk, chunk)])

    return dispatch


def _ffn_kernel(be_ref, xs_ref, sw_ref, w1_ref, w2_ref, ys_ref):
    xb = xs_ref[...].astype(jnp.bfloat16)
    w1b = w1_ref[0].astype(jnp.bfloat16)
    h = lax.dot_general(xb, w1b, (((1,), (1,)), ((), ())),
                        preferred_element_type=jnp.float32)
    xpart = h[:, :FF_DIM]
    gate = h[:, FF_DIM:]
    act = (xpart * (gate * jax.nn.sigmoid(gate))).astype(jnp.bfloat16)
    w2b = w2_ref[0].astype(jnp.bfloat16)
    y = lax.dot_general(act, w2b, (((1,), (1,)), ((), ())),
                        preferred_element_type=jnp.float32)
    sw2 = sw_ref[...].reshape(1, BLK)
    ri = lax.broadcasted_iota(jnp.int32, (BLK, BLK), 0)
    ci = lax.broadcasted_iota(jnp.int32, (BLK, BLK), 1)
    diag = jnp.where(ri == ci, jnp.broadcast_to(sw2, (BLK, BLK)), 0.0)
    ys_ref[...] = lax.dot_general(diag, y, (((1,), (0,)), ((), ())),
                                  preferred_element_type=jnp.float32)


def _ffn_call(bexp40, xs, sw3, W1, W2):
    grid_spec = pltpu.PrefetchScalarGridSpec(
        num_scalar_prefetch=1,
        grid=(N_BLOCKS,),
        in_specs=[
            pl.BlockSpec((BLK, D_MODEL), lambda b, be: (b, 0)),
            pl.BlockSpec((1, 1, BLK), lambda b, be: (b, 0, 0)),
            pl.BlockSpec((1, 2 * FF_DIM, D_MODEL), lambda b, be: (be[b], 0, 0)),
            pl.BlockSpec((1, D_MODEL, FF_DIM), lambda b, be: (be[b], 0, 0)),
        ],
        out_specs=pl.BlockSpec((BLK, D_MODEL), lambda b, be: (b, 0)),
    )
    return pl.pallas_call(
        _ffn_kernel,
        grid_spec=grid_spec,
        out_shape=jax.ShapeDtypeStruct((PADDED, D_MODEL), jnp.float32),
        compiler_params=pltpu.CompilerParams(
            dimension_semantics=("arbitrary",),
        ),
    )(bexp40, xs, sw3, W1, W2)


@functools.lru_cache(maxsize=None)
def _make_combine():
    mesh = plsc.VectorSubcoreMesh(core_axis_name="c", subcore_axis_name="s", num_cores=2, num_subcores=16)
    n_tiles = 32
    tok_per_tile = N_TOK // n_tiles

    @functools.partial(
        pl.kernel,
        out_type=jax.ShapeDtypeStruct((N_TOK, D_MODEL), jnp.float32),
        mesh=mesh,
        scratch_types=[
            pltpu.VMEM((tok_per_tile,), jnp.int32),
            pltpu.VMEM((tok_per_tile,), jnp.int32),
            pltpu.VMEM((tok_per_tile, D_MODEL), jnp.float32),
            pltpu.VMEM((tok_per_tile, D_MODEL), jnp.float32),
            pltpu.SemaphoreType.DMA,
        ],
        compiler_params=pltpu.CompilerParams(needs_layout_passes=False),
    )
    def combine(ys_hbm, pos0_hbm, pos1_hbm, out_hbm,
                i0_v, i1_v, r0_v, r1_v, sem):
        c = lax.axis_index("c")
        s = lax.axis_index("s")
        g0 = (s * 2 + c) * tok_per_tile
        pltpu.sync_copy(pos0_hbm.at[pl.ds(g0, tok_per_tile)], i0_v)
        pltpu.sync_copy(pos1_hbm.at[pl.ds(g0, tok_per_tile)], i1_v)
        pltpu.async_copy(ys_hbm.at[i0_v], r0_v, sem).wait()
        pltpu.async_copy(ys_hbm.at[i1_v], r1_v, sem).wait()

        def addrow(i, carry):
            for k in range(D_MODEL // 16):
                plsc.addupdate(r0_v.at[i, pl.ds(k * 16, 16)],
                               r1_v[i, pl.ds(k * 16, 16)])
            return carry
        lax.fori_loop(0, tok_per_tile, addrow, 0)

        pltpu.sync_copy(r0_v, out_hbm.at[pl.ds(g0, tok_per_tile)])

    return combine


@functools.partial(jax.jit, static_argnames=())
def kernel(x, W_router, W1, W2):
    B, T, C = x.shape
    flat = x.reshape(-1, C)
    pos0, pos1, w0, w1, bexp = _router_call(flat, W_router)
    pos0f = pos0.reshape(N_TOK)
    pos1f = pos1.reshape(N_TOK)
    xs, _stok, sw = _make_dispatch()(flat, pos0f, pos1f,
                                     w0.reshape(N_TOK), w1.reshape(N_TOK))
    sw3 = sw.reshape(2 * N_BLOCKS, 1, BLK)
    ys = _ffn_call(bexp.reshape(64)[:N_BLOCKS], xs, sw3, W1, W2)
    out = _make_combine()(ys, pos0f, pos1f)
    return out.reshape(B, T, C)


# speedup vs baseline: 3.1824x; 3.1824x over previous
import functools

import jax
import jax.numpy as jnp
from jax.experimental import pallas as pl
from jax.experimental.pallas import tpu as pltpu

D_MODEL = 768
FF_DIM = 512
N_EXPERTS = 8
TOP_K = 2
T_BLK = 2048


def _moe_block_kernel(x_ref, wr_ref, w1_ref, w2_ref, out_ref, rw_ref, xb_ref):
    e = pl.program_id(1)

    @pl.when(e == 0)
    def _compute_router():
        x = x_ref[...]
        xb_ref[...] = x.astype(jnp.bfloat16)
        logits = jax.lax.dot_general(
            x, wr_ref[...], (((1,), (1,)), ((), ())),
            preferred_element_type=jnp.float32)
        col = jax.lax.broadcasted_iota(jnp.int32, logits.shape, 1)
        m0 = jnp.max(logits, axis=1, keepdims=True)
        is0 = logits == m0
        i0 = jnp.min(jnp.where(is0, col, N_EXPERTS), axis=1, keepdims=True)
        oh0 = col == i0
        masked = jnp.where(oh0, -jnp.inf, logits)
        m1 = jnp.max(masked, axis=1, keepdims=True)
        is1 = masked == m1
        i1 = jnp.min(jnp.where(is1, col, N_EXPERTS), axis=1, keepdims=True)
        oh1 = col == i1
        w0 = 1.0 / (1.0 + jnp.exp(m1 - m0))
        rw_ref[...] = jnp.where(oh0, w0, 0.0) + jnp.where(oh1, 1.0 - w0, 0.0)

    xb = xb_ref[...]
    w1b = w1_ref[0].astype(jnp.bfloat16)
    w2b = w2_ref[0].astype(jnp.bfloat16)
    HF = FF_DIM // 2
    o = None
    for ci in range(2):
        xp = jax.lax.dot_general(
            xb, w1b[ci * HF:(ci + 1) * HF], (((1,), (1,)), ((), ())),
            preferred_element_type=jnp.float32)
        g = jax.lax.dot_general(
            xb, w1b[FF_DIM + ci * HF:FF_DIM + (ci + 1) * HF],
            (((1,), (1,)), ((), ())),
            preferred_element_type=jnp.float32)
        act = (xp * (g * jax.nn.sigmoid(g))).astype(jnp.bfloat16)
        oc = jax.lax.dot_general(
            act, w2b[:, ci * HF:(ci + 1) * HF], (((1,), (1,)), ((), ())),
            preferred_element_type=jnp.float32)
        o = oc if o is None else o + oc
    rw = rw_ref[...]
    ecol = jax.lax.broadcasted_iota(jnp.int32, rw.shape, 1)
    rw_e = jnp.sum(jnp.where(ecol == e, rw, 0.0), axis=1, keepdims=True)
    scaled = rw_e * o

    @pl.when(e == 0)
    def _init():
        out_ref[...] = scaled

    @pl.when(e != 0)
    def _acc():
        out_ref[...] += scaled


@functools.partial(jax.jit, static_argnames=())
def kernel(x, W_router, W1, W2):
    B, T, C = x.shape
    flat = x.reshape(-1, C)
    n_tok = flat.shape[0]
    grid = (n_tok // T_BLK, N_EXPERTS)
    out = pl.pallas_call(
        _moe_block_kernel,
        grid=grid,
        in_specs=[
            pl.BlockSpec((T_BLK, C), lambda t, e: (t, 0)),
            pl.BlockSpec((N_EXPERTS, C), lambda t, e: (0, 0)),
            pl.BlockSpec((1, 2 * FF_DIM, C), lambda t, e: (e, 0, 0)),
            pl.BlockSpec((1, C, FF_DIM), lambda t, e: (e, 0, 0)),
        ],
        out_specs=pl.BlockSpec((T_BLK, C), lambda t, e: (t, 0)),
        out_shape=jax.ShapeDtypeStruct((n_tok, C), jnp.float32),
        scratch_shapes=[
            pltpu.VMEM((T_BLK, N_EXPERTS), jnp.float32),
            pltpu.VMEM((T_BLK, C), jnp.bfloat16),
        ],
        compiler_params=pltpu.CompilerParams(
            dimension_semantics=("arbitrary", "arbitrary"),
        ),
    )(flat, W_router, W1, W2)
    return out.reshape(B, T, C)


# speedup vs baseline: 3.3467x; 1.0516x over previous
import functools

import jax
import jax.numpy as jnp
from jax.experimental import pallas as pl
from jax.experimental.pallas import tpu as pltpu

D_MODEL = 768
FF_DIM = 512
N_EXPERTS = 8
TOP_K = 2
T_BLK = 2048


def _moe_block_kernel(x_ref, wr_ref, w1_ref, w2_ref, out_ref, rw_ref, xb_ref):
    e = pl.program_id(1)

    @pl.when(e == 0)
    def _compute_router():
        x = x_ref[...]
        xb_ref[...] = x.astype(jnp.bfloat16)
        logits = jax.lax.dot_general(
            x, wr_ref[...], (((1,), (1,)), ((), ())),
            preferred_element_type=jnp.float32)
        col = jax.lax.broadcasted_iota(jnp.int32, logits.shape, 1)
        m0 = jnp.max(logits, axis=1, keepdims=True)
        is0 = logits == m0
        i0 = jnp.min(jnp.where(is0, col, N_EXPERTS), axis=1, keepdims=True)
        oh0 = col == i0
        masked = jnp.where(oh0, -jnp.inf, logits)
        m1 = jnp.max(masked, axis=1, keepdims=True)
        is1 = masked == m1
        i1 = jnp.min(jnp.where(is1, col, N_EXPERTS), axis=1, keepdims=True)
        oh1 = col == i1
        w0 = 1.0 / (1.0 + jnp.exp(m1 - m0))
        rw_ref[...] = jnp.where(oh0, w0, 0.0) + jnp.where(oh1, 1.0 - w0, 0.0)

    xb = xb_ref[...]
    w1b = w1_ref[0].astype(jnp.bfloat16)
    h = jax.lax.dot_general(
        xb, w1b, (((1,), (1,)), ((), ())),
        preferred_element_type=jnp.float32)
    xpart = h[:, :FF_DIM]
    gate = h[:, FF_DIM:]
    act = (xpart * (gate * jax.nn.sigmoid(gate))).astype(jnp.bfloat16)
    w2b = w2_ref[0].astype(jnp.bfloat16)
    o = jax.lax.dot_general(
        act, w2b, (((1,), (1,)), ((), ())),
        preferred_element_type=jnp.float32)
    rw = rw_ref[...]
    ecol = jax.lax.broadcasted_iota(jnp.int32, rw.shape, 1)
    rw_e = jnp.sum(jnp.where(ecol == e, rw, 0.0), axis=1, keepdims=True)
    scaled = rw_e * o

    @pl.when(e == 0)
    def _init():
        out_ref[...] = scaled

    @pl.when(e != 0)
    def _acc():
        out_ref[...] += scaled


@functools.partial(jax.jit, static_argnames=())
def kernel(x, W_router, W1, W2):
    B, T, C = x.shape
    flat = x.reshape(-1, C)
    n_tok = flat.shape[0]
    grid = (n_tok // T_BLK, N_EXPERTS)
    out = pl.pallas_call(
        _moe_block_kernel,
        grid=grid,
        in_specs=[
            pl.BlockSpec((T_BLK, C), lambda t, e: (t, 0)),
            pl.BlockSpec((N_EXPERTS, C), lambda t, e: (0, 0)),
            pl.BlockSpec((1, 2 * FF_DIM, C), lambda t, e: (e, 0, 0)),
            pl.BlockSpec((1, C, FF_DIM), lambda t, e: (e, 0, 0)),
        ],
        out_specs=pl.BlockSpec((T_BLK, C), lambda t, e: (t, 0)),
        out_shape=jax.ShapeDtypeStruct((n_tok, C), jnp.float32),
        scratch_shapes=[
            pltpu.VMEM((T_BLK, N_EXPERTS), jnp.float32),
            pltpu.VMEM((T_BLK, C), jnp.bfloat16),
        ],
        compiler_params=pltpu.CompilerParams(
            dimension_semantics=("arbitrary", "arbitrary"),
        ),
    )(flat, W_router, W1, W2)
    return out.reshape(B, T, C)


# speedup vs baseline: 3.3637x; 1.0051x over previous
import functools

import jax
import jax.numpy as jnp
from jax.experimental import pallas as pl
from jax.experimental.pallas import tpu as pltpu

D_MODEL = 768
FF_DIM = 512
N_EXPERTS = 8
TOP_K = 2
T_BLK = 2048


def _moe_block_kernel(x_ref, wr_ref, w1_ref, w2_ref, out_ref, rw_ref, xb_ref):
    e = pl.program_id(1)

    @pl.when(e == 0)
    def _compute_router():
        x = x_ref[...]
        xb_ref[...] = x.astype(jnp.bfloat16)
        logits = jax.lax.dot_general(
            x, wr_ref[...], (((1,), (1,)), ((), ())),
            preferred_element_type=jnp.float32)
        col = jax.lax.broadcasted_iota(jnp.int32, logits.shape, 1)
        m0 = jnp.max(logits, axis=1, keepdims=True)
        is0 = logits == m0
        i0 = jnp.min(jnp.where(is0, col, N_EXPERTS), axis=1, keepdims=True)
        oh0 = col == i0
        masked = jnp.where(oh0, -jnp.inf, logits)
        m1 = jnp.max(masked, axis=1, keepdims=True)
        is1 = masked == m1
        i1 = jnp.min(jnp.where(is1, col, N_EXPERTS), axis=1, keepdims=True)
        oh1 = col == i1
        w0 = 1.0 / (1.0 + jnp.exp(m1 - m0))
        rw_ref[...] = jnp.where(oh0, w0, 0.0) + jnp.where(oh1, 1.0 - w0, 0.0)

    xb = xb_ref[...]
    w1b = jnp.swapaxes(w1_ref[0].astype(jnp.bfloat16), 0, 1)
    h = jax.lax.dot_general(
        xb, w1b, (((1,), (0,)), ((), ())),
        preferred_element_type=jnp.float32)
    xpart = h[:, :FF_DIM]
    gate = h[:, FF_DIM:]
    act = (xpart * (gate * jax.nn.sigmoid(gate))).astype(jnp.bfloat16)
    w2b = jnp.swapaxes(w2_ref[0].astype(jnp.bfloat16), 0, 1)
    o = jax.lax.dot_general(
        act, w2b, (((1,), (0,)), ((), ())),
        preferred_element_type=jnp.float32)
    rw = rw_ref[...]
    ecol = jax.lax.broadcasted_iota(jnp.int32, rw.shape, 1)
    rw_e = jnp.sum(jnp.where(ecol == e, rw, 0.0), axis=1, keepdims=True)
    scaled = rw_e * o

    @pl.when(e == 0)
    def _init():
        out_ref[...] = scaled

    @pl.when(e != 0)
    def _acc():
        out_ref[...] += scaled


@functools.partial(jax.jit, static_argnames=())
def kernel(x, W_router, W1, W2):
    B, T, C = x.shape
    flat = x.reshape(-1, C)
    n_tok = flat.shape[0]
    grid = (n_tok // T_BLK, N_EXPERTS)
    out = pl.pallas_call(
        _moe_block_kernel,
        grid=grid,
        in_specs=[
            pl.BlockSpec((T_BLK, C), lambda t, e: (t, 0)),
            pl.BlockSpec((N_EXPERTS, C), lambda t, e: (0, 0)),
            pl.BlockSpec((1, 2 * FF_DIM, C), lambda t, e: (e, 0, 0)),
            pl.BlockSpec((1, C, FF_DIM), lambda t, e: (e, 0, 0)),
        ],
        out_specs=pl.BlockSpec((T_BLK, C), lambda t, e: (t, 0)),
        out_shape=jax.ShapeDtypeStruct((n_tok, C), jnp.float32),
        scratch_shapes=[
            pltpu.VMEM((T_BLK, N_EXPERTS), jnp.float32),
            pltpu.VMEM((T_BLK, C), jnp.bfloat16),
        ],
        compiler_params=pltpu.CompilerParams(
            dimension_semantics=("arbitrary", "arbitrary"),
        ),
    )(flat, W_router, W1, W2)
    return out.reshape(B, T, C)
